# baseline (speedup 1.0000x reference)
import functools

import jax
import jax.numpy as jnp
from jax import lax
from jax.experimental import pallas as pl
from jax.experimental.pallas import tpu as pltpu
from jax.experimental.pallas import tpu_sc as plsc

TIME = 4096
FREQ = 513
CHUNK = 20
N_WIN = TIME - CHUNK

_mesh = plsc.VectorSubcoreMesh(core_axis_name="c", subcore_axis_name="s")


@functools.partial(
    pl.kernel,
    mesh=_mesh,
    out_type=(
        jax.ShapeDtypeStruct((CHUNK, FREQ, N_WIN), jnp.float32),
        jax.ShapeDtypeStruct((FREQ, N_WIN), jnp.float32),
    ),
    scratch_types=[
        pltpu.VMEM((9, TIME), jnp.float32),
        pltpu.VMEM((9, N_WIN), jnp.float32),
    ],
    compiler_params=pltpu.CompilerParams(needs_layout_passes=False),
)
def _sc_all(mt_ref, vt_ref, x_ref, y_ref, in_s, out_s):
    w = lax.axis_index("s") * 2 + lax.axis_index("c")
    lanes = lax.iota(jnp.int32, 16)

    def assemble(nrows, shift):
        for s in range(nrows):
            rows = jnp.full((16,), s, jnp.int32)

            @plsc.parallel_loop(0, N_WIN - 12, 16)
            def _(l0):
                cols = l0 + shift + lanes
                out_s[s, pl.ds(l0, 16)] = plsc.load_gather(in_s, [rows, cols])

            tail = N_WIN - 16
            out_s[s, pl.ds(tail, 16)] = plsc.load_gather(
                in_s, [rows, tail + shift + lanes])

    def do_task(t, nrows):
        r0 = pl.multiple_of(8 * t, 8) if nrows == 8 else 504
        in_dst = in_s if nrows == 9 else in_s.at[pl.ds(0, 8), :]
        out_src = out_s if nrows == 9 else out_s.at[pl.ds(0, 8), :]

        pltpu.sync_copy(mt_ref.at[pl.ds(r0, nrows), :], in_dst)

        def cbody(c, carry):
            assemble(nrows, c)
            pltpu.sync_copy(out_src, x_ref.at[c, pl.ds(r0, nrows), :])
            return carry

        lax.fori_loop(0, CHUNK, cbody, 0)

        pltpu.sync_copy(vt_ref.at[pl.ds(r0, nrows), :], in_dst)
        assemble(nrows, CHUNK)
        pltpu.sync_copy(out_src, y_ref.at[pl.ds(r0, nrows), :])

    do_task(w, 8)

    @pl.when(w < 31)
    def _():
        do_task(w + 32, 8)

    @pl.when(w == 31)
    def _():
        do_task(63, 9)


def kernel(mixed_mag, vocal_mag):
    mt = mixed_mag.T
    vt = vocal_mag.T
    x_alt, y_alt = _sc_all(mt, vt)
    return x_alt.transpose(2, 1, 0), y_alt.T


# speedup vs baseline: 5.3469x; 5.3469x over previous
import jax
import jax.numpy as jnp
from jax.experimental import pallas as pl
from jax.experimental.pallas import tpu as pltpu

TIME = 4096
FREQ = 513
CHUNK = 20
N_WIN = TIME - CHUNK
JB = 256
NJ = (N_WIN + JB - 1) // JB
PADW = TIME + 128


def _body(mt_hbm, vt_hbm, x_ref, y_ref, mscr, vscr, sem0, sem1):
    jb = pl.program_id(0)

    @pl.when(jb == 0)
    def _():
        cp0 = pltpu.make_async_copy(mt_hbm, mscr.at[:, pl.ds(0, TIME)], sem0)
        cp1 = pltpu.make_async_copy(vt_hbm, vscr.at[:, pl.ds(0, TIME)], sem1)
        cp0.start()
        cp1.start()
        cp0.wait()
        cp1.wait()

    base = pl.multiple_of(jb * JB, 128)
    w = mscr[:, pl.ds(base, JB + 128)]
    for c in range(CHUNK):
        x_ref[c, :, :] = w[:, c:c + JB]
    wv = vscr[:, pl.ds(base, JB + 128)]
    y_ref[...] = wv[:, CHUNK:CHUNK + JB]


_call = pl.pallas_call(
    _body,
    grid=(NJ,),
    in_specs=[
        pl.BlockSpec(memory_space=pl.ANY),
        pl.BlockSpec(memory_space=pl.ANY),
    ],
    out_specs=[
        pl.BlockSpec((CHUNK, FREQ, JB), lambda j: (0, 0, j)),
        pl.BlockSpec((FREQ, JB), lambda j: (0, j)),
    ],
    out_shape=[
        jax.ShapeDtypeStruct((CHUNK, FREQ, N_WIN), jnp.float32),
        jax.ShapeDtypeStruct((FREQ, N_WIN), jnp.float32),
    ],
    scratch_shapes=[
        pltpu.VMEM((FREQ, PADW), jnp.float32),
        pltpu.VMEM((FREQ, PADW), jnp.float32),
        pltpu.SemaphoreType.DMA,
        pltpu.SemaphoreType.DMA,
    ],
    compiler_params=pltpu.CompilerParams(vmem_limit_bytes=58 * 1024 * 1024),
)


def kernel(mixed_mag, vocal_mag):
    mt = mixed_mag.T
    vt = vocal_mag.T
    x_alt, y_alt = _call(mt, vt)
    return x_alt.transpose(2, 1, 0), y_alt.T
